# baseline (speedup 1.0000x reference)
import functools

import jax
import jax.numpy as jnp
from jax import lax
from jax.experimental import pallas as pl
from jax.experimental.pallas import tpu as pltpu
from jax.experimental.pallas import tpu_sc as plsc

_B = 16384
_D = 64

_info = plsc.get_sparse_core_info()
_NC = _info.num_cores
_NS = _info.num_subcores
_L = _info.num_lanes
_NW = _NC * _NS
_FG = 8
_NFG = _D // _FG
_NBG = _NW // _NFG
_B_PER_W = _B // _NBG
_HALF = _B_PER_W // 2
_UNROLL = 4

_mesh = plsc.VectorSubcoreMesh(core_axis_name="c", subcore_axis_name="s")


@functools.partial(
    pl.kernel,
    mesh=_mesh,
    out_type=jax.ShapeDtypeStruct((_D, _B), jnp.float32),
    scratch_types=[
        pltpu.VMEM((_B_PER_W,), jnp.int32),
        pltpu.VMEM((2, _D), jnp.float32),
        pltpu.VMEM((_FG, _HALF), jnp.float32),
        pltpu.VMEM((_FG, _HALF), jnp.float32),
        pltpu.SemaphoreType.DMA,
        pltpu.SemaphoreType.DMA,
        pltpu.SemaphoreType.DMA,
    ],
)
def _embed_lookup_t(
    idx_hbm, table_hbm, out_hbm, idx_v, table_v, buf0, buf1, s0, s1, si
):
    wid = lax.axis_index("s") * _NC + lax.axis_index("c")
    fg = wid >> 2
    bg = wid & 3
    cpi = pltpu.async_copy(
        idx_hbm.at[pl.ds(bg * _B_PER_W, _B_PER_W)], idx_v, si
    )
    pltpu.sync_copy(table_hbm, table_v)

    lanes = lax.iota(jnp.int32, _L)
    zeros = lanes * 0
    chunk = (fg >> 1) * _L
    t0c = table_v[0, pl.ds(chunk, _L)]
    t1c = table_v[1, pl.ds(chunk, _L)]
    t0b, dtb = [], []
    for k in range(_FG):
        sel = zeros + ((fg * _FG + k) & (_L - 1))
        t0 = t0c.at[sel].get(mode="promise_in_bounds")
        t1 = t1c.at[sel].get(mode="promise_in_bounds")
        t0b.append(t0)
        dtb.append(t1 - t0)
    cpi.wait()

    def make_body(buf, idx_base, buf_base):
        def body(j, carry):
            for u in range(_UNROLL):
                off = buf_base + j * _UNROLL * _L + u * _L
                lf = idx_v[pl.ds(idx_base + off, _L)].astype(jnp.float32)
                for k in range(_FG):
                    buf[k, pl.ds(off, _L)] = t0b[k] + lf * dtb[k]
            return carry
        return body

    row = pl.ds(fg * _FG, _FG)
    q = _HALF // 2
    iters = q // (_UNROLL * _L)
    cps = []
    for c in range(4):
        buf = (buf0, buf1)[c & 1]
        half = (c >> 1) * _HALF
        lax.fori_loop(0, iters, make_body(buf, half, (c & 1) * q), 0)
        if len(cps) >= 2:
            cps[c - 2].wait()
        cps.append(
            pltpu.async_copy(
                buf.at[:, pl.ds((c & 1) * q, q)],
                out_hbm.at[row, pl.ds(bg * _B_PER_W + half + (c & 1) * q, q)],
                (s0, s1)[c & 1],
            )
        )
    cps[2].wait()
    cps[3].wait()


def kernel(layer, table):
    return _embed_lookup_t(layer, table).T
